# baseline (speedup 1.0000x reference)
import jax
import jax.numpy as jnp
from jax import lax
from jax.experimental import pallas as pl
from jax.experimental.pallas import tpu as pltpu
from jax.experimental.pallas import tpu_sc as plsc

B, C = 16384, 1000
NW = 32
RPW = B // NW
CH = 32
NCHUNK = RPW // CH
GPC = CH // 16


def _cw_body(pred_hbm, y_hbm, out_hbm, buf0, buf1, y_v, out_v, sem0, sem1):
    cid = lax.axis_index("c")
    sid = lax.axis_index("s")
    wid = sid * 2 + cid
    row0 = wid * RPW
    pltpu.sync_copy(y_hbm.at[pl.ds(row0, RPW)], y_v)

    bufs = [buf0, buf1]
    sems = [sem0, sem1]
    neg_inf = jnp.full((16,), -jnp.inf, jnp.float32)
    lane = lax.iota(jnp.int32, 16)
    zero16 = jnp.zeros((16,), jnp.int32)

    copies = [None, None]
    copies[0] = pltpu.async_copy(
        pred_hbm.at[pl.ds(row0, CH)], bufs[0], sems[0])
    for c in range(NCHUNK):
        if c + 1 < NCHUNK:
            nb = (c + 1) % 2
            copies[nb] = pltpu.async_copy(
                pred_hbm.at[pl.ds(row0 + (c + 1) * CH, CH)],
                bufs[nb], sems[nb])
        cb = c % 2
        copies[cb].wait()
        buf = bufs[cb]
        for g in range(GPC):
            gg = c * GPC + g
            yv = y_v[pl.ds(gg * 16, 16)]
            rowv = g * 16 + lane
            class_pred = plsc.load_gather(buf, [rowv, yv])
            plsc.store_scatter(buf, [rowv, yv], neg_inf)

            def body(i, carry):
                a0, a1, cv = carry
                for _ in range(4):
                    v0 = plsc.load_gather(buf, [rowv, cv])
                    v1 = plsc.load_gather(buf, [rowv, cv + 1])
                    a0 = jnp.maximum(a0, v0)
                    a1 = jnp.maximum(a1, v1)
                    cv = cv + 2
                return a0, a1, cv

            acc0, acc1, _ = lax.fori_loop(
                0, C // 8, body, (neg_inf, neg_inf, zero16))
            loss = jnp.maximum(acc0, acc1) - class_pred
            out_v[pl.ds(gg * 16, 16)] = loss

    pltpu.sync_copy(out_v, out_hbm.at[pl.ds(row0, RPW)])


_run = pl.kernel(
    _cw_body,
    out_type=jax.ShapeDtypeStruct((B,), jnp.float32),
    mesh=plsc.VectorSubcoreMesh(core_axis_name="c", subcore_axis_name="s"),
    scratch_types=[
        pltpu.VMEM((CH, C), jnp.float32),
        pltpu.VMEM((CH, C), jnp.float32),
        pltpu.VMEM((RPW,), jnp.int32),
        pltpu.VMEM((RPW,), jnp.float32),
        pltpu.SemaphoreType.DMA,
        pltpu.SemaphoreType.DMA,
    ],
    compiler_params=pltpu.CompilerParams(needs_layout_passes=False),
)


@jax.jit
def kernel(pred, y):
    return _run(pred, y.astype(jnp.int32))


# speedup vs baseline: 1.7970x; 1.7970x over previous
import jax
import jax.numpy as jnp
from jax import lax
from jax.experimental import pallas as pl
from jax.experimental.pallas import tpu as pltpu
from jax.experimental.pallas import tpu_sc as plsc

B, C = 16384, 1000
NW = 32
RPW = B // NW
CH = 32
NCHUNK = RPW // CH
GPC = CH // 16


def _cw_body(pred_hbm, y_hbm, out_hbm, buf0, buf1, y_v, out_v, sem0, sem1):
    cid = lax.axis_index("c")
    sid = lax.axis_index("s")
    wid = sid * 2 + cid
    row0 = wid * RPW
    pltpu.sync_copy(y_hbm.at[pl.ds(row0, RPW)], y_v)

    bufs = [buf0, buf1]
    sems = [sem0, sem1]
    neg_inf = jnp.full((16,), -jnp.inf, jnp.float32)
    lane = lax.iota(jnp.int32, 16)
    loff = lane * 8

    copies = [None, None]
    copies[0] = pltpu.async_copy(
        pred_hbm.at[pl.ds(row0, CH)], bufs[0], sems[0])
    for c in range(NCHUNK):
        if c + 1 < NCHUNK:
            nb = (c + 1) % 2
            copies[nb] = pltpu.async_copy(
                pred_hbm.at[pl.ds(row0 + (c + 1) * CH, CH)],
                bufs[nb], sems[nb])
        cb = c % 2
        copies[cb].wait()
        buf = bufs[cb]
        for g in range(GPC):
            gg = c * GPC + g
            yv = y_v[pl.ds(gg * 16, 16)]
            rowv = g * 16 + lane
            class_pred = plsc.load_gather(buf, [rowv, yv])
            plsc.store_scatter(buf, [rowv, yv], neg_inf)

            def body(i, carry):
                a0, a1, cv0, cv1 = carry
                for _ in range(4):
                    v0 = plsc.load_gather(buf, [rowv, cv0])
                    v1 = plsc.load_gather(buf, [rowv, cv1])
                    a0 = jnp.maximum(a0, v0)
                    a1 = jnp.maximum(a1, v1)
                    cv0 = cv0 + 2
                    cv0 = jnp.where(cv0 >= C, cv0 - C, cv0)
                    cv1 = cv1 + 2
                    cv1 = jnp.where(cv1 >= C, cv1 - C, cv1)
                return a0, a1, cv0, cv1

            cv1_init = loff + 1
            acc0, acc1, _, _ = lax.fori_loop(
                0, C // 8, body, (neg_inf, neg_inf, loff, cv1_init))
            loss = jnp.maximum(acc0, acc1) - class_pred
            out_v[pl.ds(gg * 16, 16)] = loss

    pltpu.sync_copy(out_v, out_hbm.at[pl.ds(row0, RPW)])


_run = pl.kernel(
    _cw_body,
    out_type=jax.ShapeDtypeStruct((B,), jnp.float32),
    mesh=plsc.VectorSubcoreMesh(core_axis_name="c", subcore_axis_name="s"),
    scratch_types=[
        pltpu.VMEM((CH, C), jnp.float32),
        pltpu.VMEM((CH, C), jnp.float32),
        pltpu.VMEM((RPW,), jnp.int32),
        pltpu.VMEM((RPW,), jnp.float32),
        pltpu.SemaphoreType.DMA,
        pltpu.SemaphoreType.DMA,
    ],
    compiler_params=pltpu.CompilerParams(needs_layout_passes=False),
)


@jax.jit
def kernel(pred, y):
    return _run(pred, y.astype(jnp.int32))


# speedup vs baseline: 1.7998x; 1.0016x over previous
import jax
import jax.numpy as jnp
from jax import lax
from jax.experimental import pallas as pl
from jax.experimental.pallas import tpu as pltpu
from jax.experimental.pallas import tpu_sc as plsc

B, C = 16384, 1000
NW = 32
RPW = B // NW
CH = 32
NCHUNK = RPW // CH
GPC = CH // 16


def _cw_body(pred_hbm, y_hbm, out_hbm, buf0, buf1, y_v, out_v, sem0, sem1):
    cid = lax.axis_index("c")
    sid = lax.axis_index("s")
    wid = sid * 2 + cid
    row0 = wid * RPW
    pltpu.sync_copy(y_hbm.at[pl.ds(row0, RPW)], y_v)

    bufs = [buf0, buf1]
    sems = [sem0, sem1]
    neg_inf = jnp.full((16,), -jnp.inf, jnp.float32)
    lane = lax.iota(jnp.int32, 16)
    loff = lane * 8

    copies = [None, None]
    copies[0] = pltpu.async_copy(
        pred_hbm.at[pl.ds(row0, CH)], bufs[0], sems[0])
    for c in range(NCHUNK):
        if c + 1 < NCHUNK:
            nb = (c + 1) % 2
            copies[nb] = pltpu.async_copy(
                pred_hbm.at[pl.ds(row0 + (c + 1) * CH, CH)],
                bufs[nb], sems[nb])
        cb = c % 2
        copies[cb].wait()
        buf = bufs[cb]
        for g in range(GPC):
            gg = c * GPC + g
            yv = y_v[pl.ds(gg * 16, 16)]
            rowv = g * 16 + lane
            class_pred = plsc.load_gather(buf, [rowv, yv])
            plsc.store_scatter(buf, [rowv, yv], neg_inf)

            def body(i, carry):
                a0, a1, cv0, cv1 = carry
                for _ in range(4):
                    v0 = plsc.load_gather(buf, [rowv, cv0])
                    v1 = plsc.load_gather(buf, [rowv, cv1])
                    a0 = jnp.maximum(a0, v0)
                    a1 = jnp.maximum(a1, v1)
                    cv0 = cv0 + 2
                    cv0 = jnp.where(cv0 >= C, cv0 - C, cv0)
                    cv1 = cv1 + 2
                    cv1 = jnp.where(cv1 >= C, cv1 - C, cv1)
                return a0, a1, cv0, cv1

            cv1_init = loff + 1
            acc0, acc1, _, _ = lax.fori_loop(
                0, C // 8, body, (neg_inf, neg_inf, loff, cv1_init))
            loss = jnp.maximum(acc0, acc1) - class_pred
            out_v[pl.ds(gg * 16, 16)] = loss

    pltpu.sync_copy(out_v, out_hbm.at[pl.ds(row0, RPW)])


_run = pl.kernel(
    _cw_body,
    out_type=jax.ShapeDtypeStruct((B,), jnp.float32),
    mesh=plsc.VectorSubcoreMesh(core_axis_name="c", subcore_axis_name="s"),
    scratch_types=[
        pltpu.VMEM((CH, C), jnp.float32),
        pltpu.VMEM((CH, C), jnp.float32),
        pltpu.VMEM((RPW,), jnp.int32),
        pltpu.VMEM((RPW,), jnp.float32),
        pltpu.SemaphoreType.DMA,
        pltpu.SemaphoreType.DMA,
    ],
    compiler_params=pltpu.CompilerParams(
        needs_layout_passes=False, skip_device_barrier=True),
)


@jax.jit
def kernel(pred, y):
    return _run(pred, y.astype(jnp.int32))


# speedup vs baseline: 2.4978x; 1.3878x over previous
import jax
import jax.numpy as jnp
from jax import lax
from jax.experimental import pallas as pl
from jax.experimental.pallas import tpu as pltpu
from jax.experimental.pallas import tpu_sc as plsc

B, C = 16384, 1000
NW = 32
RPW = B // NW
CH = 32
NCHUNK = RPW // CH
GPC = CH // 16
NLOAD = C // 16
TAIL0 = C - 16


def _cw_body(pred_hbm, y_hbm, out_hbm, buf0, buf1, y_v, out_v, scr,
             sem0, sem1):
    cid = lax.axis_index("c")
    sid = lax.axis_index("s")
    wid = sid * 2 + cid
    row0 = wid * RPW
    pltpu.sync_copy(y_hbm.at[pl.ds(row0, RPW)], y_v)

    bufs = [buf0, buf1]
    sems = [sem0, sem1]
    neg_inf = jnp.full((16,), -jnp.inf, jnp.float32)
    lane = lax.iota(jnp.int32, 16)
    last = jnp.full((16,), 15, jnp.int32)

    copies = [None, None]
    copies[0] = pltpu.async_copy(
        pred_hbm.at[pl.ds(row0, CH)], bufs[0], sems[0])
    for c in range(NCHUNK):
        if c + 1 < NCHUNK:
            nb = (c + 1) % 2
            copies[nb] = pltpu.async_copy(
                pred_hbm.at[pl.ds(row0 + (c + 1) * CH, CH)],
                bufs[nb], sems[nb])
        cb = c % 2
        copies[cb].wait()
        buf = bufs[cb]
        for g in range(GPC):
            gg = c * GPC + g
            yv = y_v[pl.ds(gg * 16, 16)]
            rowv = g * 16 + lane
            class_pred = plsc.load_gather(buf, [rowv, yv])
            plsc.store_scatter(buf, [rowv, yv], neg_inf)

            def row_pair(i, carry):
                for u in range(2):
                    r = g * 16 + i * 2 + u
                    accs = [neg_inf, neg_inf, neg_inf, neg_inf]
                    for k in range(NLOAD):
                        accs[k % 4] = jnp.maximum(
                            accs[k % 4], buf[r, pl.ds(16 * k, 16)])
                    accs[2] = jnp.maximum(accs[2], buf[r, pl.ds(TAIL0, 16)])
                    comb = jnp.maximum(jnp.maximum(accs[0], accs[1]),
                                       jnp.maximum(accs[2], accs[3]))
                    scr[i * 2 + u, pl.ds(0, 16)] = plsc.cummax(comb)
                return carry

            lax.fori_loop(0, 8, row_pair, 0)
            rowmax = plsc.load_gather(scr, [lane, last])
            out_v[pl.ds(gg * 16, 16)] = rowmax - class_pred

    pltpu.sync_copy(out_v, out_hbm.at[pl.ds(row0, RPW)])


_run = pl.kernel(
    _cw_body,
    out_type=jax.ShapeDtypeStruct((B,), jnp.float32),
    mesh=plsc.VectorSubcoreMesh(core_axis_name="c", subcore_axis_name="s"),
    scratch_types=[
        pltpu.VMEM((CH, C), jnp.float32),
        pltpu.VMEM((CH, C), jnp.float32),
        pltpu.VMEM((RPW,), jnp.int32),
        pltpu.VMEM((RPW,), jnp.float32),
        pltpu.VMEM((16, 16), jnp.float32),
        pltpu.SemaphoreType.DMA,
        pltpu.SemaphoreType.DMA,
    ],
    compiler_params=pltpu.CompilerParams(needs_layout_passes=False),
)


@jax.jit
def kernel(pred, y):
    return _run(pred, y.astype(jnp.int32))
